# Initial kernel scaffold; baseline (speedup 1.0000x reference)
import jax
import jax.numpy as jnp
from jax.experimental import pallas as pl


def kernel(emb, edge_index, etype, V, coeff, W_loop):
    raise NotImplementedError("write your pallas kernel here")


# speedup vs baseline: 9.1688x; 9.1688x over previous
import functools

import jax
import jax.numpy as jnp
from jax import lax
from jax.experimental import pallas as pl
from jax.experimental.pallas import tpu as pltpu
from jax.experimental.pallas import tpu_sc as plsc

N = 10000
E = 320000
D = 128
R = 8
NB = 4
R1 = R + 1

NC = 2
NS = 16
NW = NC * NS
EPW = E // NW
C = 80
NCHUNK = EPW // C
RSTRIDE = 624
RSIZE = 640

BN = 1000
NBLK = N // BN


def _build_w(coeff, V, W_loop):
    def body(coeff_ref, v_ref, wl_ref, o_ref):
        for r in range(R):
            acc = coeff_ref[r, 0] * v_ref[0]
            for b in range(1, NB):
                acc = acc + coeff_ref[r, b] * v_ref[b]
            o_ref[r] = acc
        o_ref[R] = wl_ref[...]

    return pl.pallas_call(
        body,
        out_shape=jax.ShapeDtypeStruct((R1, D, D), jnp.float32),
        in_specs=[
            pl.BlockSpec(memory_space=pltpu.SMEM),
            pl.BlockSpec(memory_space=pltpu.VMEM),
            pl.BlockSpec(memory_space=pltpu.VMEM),
        ],
        out_specs=pl.BlockSpec(memory_space=pltpu.VMEM),
    )(coeff, V, W_loop)


def _transform(xs, W):
    n_in = len(xs)

    def body(*refs):
        x_refs = refs[:n_in]
        w_ref = refs[n_in]
        o_ref = refs[n_in + 1]
        x = x_refs[0][...]
        for xr in x_refs[1:]:
            x = x + xr[...]
        o_ref[...] = jnp.dot(x, w_ref[0], preferred_element_type=jnp.float32)

    in_specs = [
        pl.BlockSpec((BN, D), functools.partial(lambda off, nb, r: (off + nb, 0), off))
        for (_, off) in xs
    ]
    in_specs.append(pl.BlockSpec((1, D, D), lambda nb, r: (r, 0, 0)))

    return pl.pallas_call(
        body,
        grid=(NBLK, R1),
        out_shape=jax.ShapeDtypeStruct((R1 * N, D), jnp.float32),
        in_specs=in_specs,
        out_specs=pl.BlockSpec((BN, D), lambda nb, r: (r * NBLK + nb, 0)),
    )(*[a for (a, _) in xs], W)


def _sc_message(Y, src, etype, dst, zeros):
    mesh = plsc.VectorSubcoreMesh(
        core_axis_name="c", subcore_axis_name="s",
        num_cores=NC, num_subcores=NS)

    @functools.partial(
        pl.kernel,
        out_type=jax.ShapeDtypeStruct((NC * N, D), jnp.float32),
        mesh=mesh,
        scratch_types=[
            pltpu.VMEM((C,), jnp.int32),
            pltpu.VMEM((C,), jnp.int32),
            pltpu.VMEM((C,), jnp.int32),
            pltpu.VMEM((C,), jnp.int32),
            pltpu.VMEM((C, D), jnp.float32),
            pltpu.VMEM_SHARED((N, D), jnp.float32),
            pltpu.SemaphoreType.DMA,
        ],
    )
    def k(y_hbm, src_hbm, et_hbm, dst_hbm, z_hbm, out_hbm,
          srcv, etv, dstv, gidxv, rowsv, agg, sem):
        c = lax.axis_index("c")
        s = lax.axis_index("s")
        wid = c * NS + s
        row0 = s * RSTRIDE
        pltpu.sync_copy(z_hbm.at[pl.ds(row0, RSIZE)], agg.at[pl.ds(row0, RSIZE)])
        plsc.subcore_barrier()

        ebase = wid * EPW

        def chunk(j, carry):
            base = ebase + j * C
            pltpu.sync_copy(src_hbm.at[pl.ds(base, C)], srcv)
            pltpu.sync_copy(et_hbm.at[pl.ds(base, C)], etv)
            pltpu.sync_copy(dst_hbm.at[pl.ds(base, C)], dstv)
            for i in range(C // 16):
                sl = pl.ds(i * 16, 16)
                gidxv[sl] = etv[sl] * N + srcv[sl]
            pltpu.async_copy(y_hbm.at[gidxv], rowsv, sem).wait()
            pltpu.sync_copy(rowsv, agg.at[dstv], add=True)
            return carry

        lax.fori_loop(0, NCHUNK, chunk, 0)
        plsc.subcore_barrier()
        pltpu.sync_copy(agg.at[pl.ds(row0, RSIZE)],
                        out_hbm.at[pl.ds(c * N + row0, RSIZE)])

    return k(Y, src, etype, dst, zeros)


def _final_add(P, Y):
    def body(p0_ref, p1_ref, l_ref, o_ref):
        o_ref[...] = p0_ref[...] + p1_ref[...] + l_ref[...]

    return pl.pallas_call(
        body,
        grid=(NBLK,),
        out_shape=jax.ShapeDtypeStruct((N, D), jnp.float32),
        in_specs=[
            pl.BlockSpec((BN, D), lambda i: (i, 0)),
            pl.BlockSpec((BN, D), lambda i: (NBLK + i, 0)),
            pl.BlockSpec((BN, D), lambda i: (R * NBLK + i, 0)),
        ],
        out_specs=pl.BlockSpec((BN, D), lambda i: (i, 0)),
    )(P, P, Y)


@jax.jit
def kernel(emb, edge_index, etype, V, coeff, W_loop):
    src = edge_index[0]
    dst = edge_index[1]
    zeros = jnp.zeros((N, D), jnp.float32)

    W = _build_w(coeff, V, W_loop)

    Y1 = _transform([(emb, 0)], W)
    P1 = _sc_message(Y1, src, etype, dst, zeros)
    Y2 = _transform(
        [(P1, 0), (P1, NBLK), (Y1, R * NBLK), (emb, 0)], W)
    P2 = _sc_message(Y2, src, etype, dst, zeros)
    return _final_add(P2, Y2)


# speedup vs baseline: 19.5898x; 2.1366x over previous
import functools

import jax
import jax.numpy as jnp
from jax import lax
from jax.experimental import pallas as pl
from jax.experimental.pallas import tpu as pltpu
from jax.experimental.pallas import tpu_sc as plsc

N = 10000
E = 320000
D = 128
R = 8
NB = 4
R1 = R + 1

NC = 2
NS = 16
NW = NC * NS
EPW = E // NW
C = 80
NCHUNK = EPW // C
RSTRIDE = 624
RSIZE = 640

BN = 1000
NBLK = N // BN


def _build_w(coeff, V, W_loop):
    def body(coeff_ref, v_ref, wl_ref, o_ref):
        for r in range(R):
            acc = coeff_ref[r, 0] * v_ref[0]
            for b in range(1, NB):
                acc = acc + coeff_ref[r, b] * v_ref[b]
            o_ref[r] = acc
        o_ref[R] = wl_ref[...]

    return pl.pallas_call(
        body,
        out_shape=jax.ShapeDtypeStruct((R1, D, D), jnp.float32),
        in_specs=[
            pl.BlockSpec(memory_space=pltpu.SMEM),
            pl.BlockSpec(memory_space=pltpu.VMEM),
            pl.BlockSpec(memory_space=pltpu.VMEM),
        ],
        out_specs=pl.BlockSpec(memory_space=pltpu.VMEM),
    )(coeff, V, W_loop)


def _transform(xs, W):
    n_in = len(xs)

    def body(*refs):
        x_refs = refs[:n_in]
        w_ref = refs[n_in]
        o_ref = refs[n_in + 1]
        x = x_refs[0][...]
        for xr in x_refs[1:]:
            x = x + xr[...]
        o_ref[...] = jnp.dot(x, w_ref[0], preferred_element_type=jnp.float32)

    in_specs = [
        pl.BlockSpec((BN, D), functools.partial(lambda off, nb, r: (off + nb, 0), off))
        for (_, off) in xs
    ]
    in_specs.append(pl.BlockSpec((1, D, D), lambda nb, r: (r, 0, 0)))

    return pl.pallas_call(
        body,
        grid=(NBLK, R1),
        out_shape=jax.ShapeDtypeStruct((R1 * N, D), jnp.float32),
        in_specs=in_specs,
        out_specs=pl.BlockSpec((BN, D), lambda nb, r: (r * NBLK + nb, 0)),
    )(*[a for (a, _) in xs], W)


def _gidx(src2, et2):
    def body(s_ref, e_ref, o_ref):
        o_ref[...] = e_ref[...] * N + s_ref[...]

    return pl.pallas_call(
        body,
        out_shape=jax.ShapeDtypeStruct(src2.shape, jnp.int32),
    )(src2, et2)


def _sc_message(Y, gidx, dst3, zeros):
    mesh = plsc.VectorSubcoreMesh(
        core_axis_name="c", subcore_axis_name="s",
        num_cores=NC, num_subcores=NS)

    @functools.partial(
        pl.kernel,
        out_type=jax.ShapeDtypeStruct((NC * N, D), jnp.float32),
        mesh=mesh,
        scratch_types=[
            pltpu.VMEM((NCHUNK, C), jnp.int32),
            pltpu.VMEM((EPW,), jnp.int32),
            pltpu.VMEM((C, D), jnp.float32),
            pltpu.VMEM((C, D), jnp.float32),
            pltpu.VMEM_SHARED((N, D), jnp.float32),
            pltpu.SemaphoreType.DMA,
            pltpu.SemaphoreType.DMA,
        ],
    )
    def k(y_hbm, gidx_hbm, dst_hbm, z_hbm, out_hbm,
          dstm, gidxv, rows0, rows1, agg, sem0, sem1):
        c = lax.axis_index("c")
        s = lax.axis_index("s")
        wid = c * NS + s
        row0 = s * RSTRIDE
        ebase = wid * EPW

        cp_gi = pltpu.async_copy(gidx_hbm.at[pl.ds(ebase, EPW)], gidxv, sem0)
        pltpu.sync_copy(z_hbm.at[pl.ds(row0, RSIZE)],
                        agg.at[pl.ds(row0, RSIZE)])
        pltpu.sync_copy(dst_hbm.at[wid], dstm)
        cp_gi.wait()

        plsc.subcore_barrier()

        cpa = pltpu.async_copy(y_hbm.at[gidxv.at[pl.ds(0, C)]], rows0, sem0)
        cpb = pltpu.async_copy(y_hbm.at[gidxv.at[pl.ds(C, C)]], rows1, sem1)

        def pair(jj, carry):
            j0 = 2 * jj
            cpa.wait()
            pltpu.sync_copy(rows0, agg.at[dstm.at[j0]], add=True)
            pltpu.async_copy(
                y_hbm.at[gidxv.at[pl.ds((j0 + 2) * C, C)]], rows0, sem0)
            cpb.wait()
            pltpu.sync_copy(rows1, agg.at[dstm.at[j0 + 1]], add=True)

            @pl.when(jj < (NCHUNK - 3) // 2)
            def _():
                pltpu.async_copy(
                    y_hbm.at[gidxv.at[pl.ds((j0 + 3) * C, C)]], rows1, sem1)
            return carry

        lax.fori_loop(0, (NCHUNK - 1) // 2, pair, 0)
        cpa.wait()
        pltpu.sync_copy(rows0, agg.at[dstm.at[NCHUNK - 1]], add=True)

        plsc.subcore_barrier()
        pltpu.sync_copy(agg.at[pl.ds(row0, RSIZE)],
                        out_hbm.at[pl.ds(c * N + row0, RSIZE)])

    return k(Y, gidx, dst3, zeros)


def _final_add(P, Y):
    def body(p0_ref, p1_ref, l_ref, o_ref):
        o_ref[...] = p0_ref[...] + p1_ref[...] + l_ref[...]

    return pl.pallas_call(
        body,
        grid=(NBLK,),
        out_shape=jax.ShapeDtypeStruct((N, D), jnp.float32),
        in_specs=[
            pl.BlockSpec((BN, D), lambda i: (i, 0)),
            pl.BlockSpec((BN, D), lambda i: (NBLK + i, 0)),
            pl.BlockSpec((BN, D), lambda i: (R * NBLK + i, 0)),
        ],
        out_specs=pl.BlockSpec((BN, D), lambda i: (i, 0)),
    )(P, P, Y)


@jax.jit
def kernel(emb, edge_index, etype, V, coeff, W_loop):
    src2 = edge_index[0].reshape(E // D, D)
    et2 = etype.reshape(E // D, D)
    dst3 = edge_index[1].reshape(NW, NCHUNK, C)
    zeros = jnp.zeros((N, D), jnp.float32)

    W = _build_w(coeff, V, W_loop)
    gidx = _gidx(src2, et2).reshape(E)

    Y1 = _transform([(emb, 0)], W)
    P1 = _sc_message(Y1, gidx, dst3, zeros)
    Y2 = _transform(
        [(P1, 0), (P1, NBLK), (Y1, R * NBLK), (emb, 0)], W)
    P2 = _sc_message(Y2, gidx, dst3, zeros)
    return _final_add(P2, Y2)


# speedup vs baseline: 20.2862x; 1.0355x over previous
import functools

import jax
import jax.numpy as jnp
from jax import lax
from jax.experimental import pallas as pl
from jax.experimental.pallas import tpu as pltpu
from jax.experimental.pallas import tpu_sc as plsc

N = 10000
E = 320000
D = 128
R = 8
NB = 4
R1 = R + 1

NC = 2
NS = 16
NW = NC * NS
EPW = E // NW
C = 80
NCHUNK = EPW // C
RSTRIDE = 624
RSIZE = 640

BN = 1000
NBLK = N // BN


GROWS = NBLK * 8
GCOLS = E // GROWS


def _transform(xs, coeff_pad, V_pad, idx2=None):
    n_in = len(xs)
    with_gidx = idx2 is not None

    def body(*refs):
        coeff_ref = refs[0]
        v_ref = refs[1]
        x_refs = refs[2:2 + n_in]
        rest = refs[2 + n_in:]
        if with_gidx:
            s_ref, e_ref, o_ref, g_ref, w_scr = rest
        else:
            (o_ref, w_scr) = rest[:2]
        nb = pl.program_id(0)
        r = pl.program_id(1)

        @pl.when(nb == 0)
        def _():
            w = coeff_ref[r, 0] * v_ref[0]
            for b in range(1, NB + 1):
                w = w + coeff_ref[r, b] * v_ref[b]
            w_scr[r] = w

        if with_gidx:
            g_ref[...] = e_ref[...] * N + s_ref[...]

        x = x_refs[0][...]
        for xr in x_refs[1:]:
            x = x + xr[...]
        o_ref[...] = jnp.dot(x, w_scr[r], preferred_element_type=jnp.float32)

    in_specs = [
        pl.BlockSpec(memory_space=pltpu.SMEM),
        pl.BlockSpec((NB + 1, D, D), lambda nb, r: (0, 0, 0)),
    ]
    in_specs += [
        pl.BlockSpec((BN, D), functools.partial(lambda off, nb, r: (off + nb, 0), off))
        for (_, off) in xs
    ]
    args = [coeff_pad, V_pad] + [a for (a, _) in xs]
    out_shape = jax.ShapeDtypeStruct((R1 * N, D), jnp.float32)
    out_specs = pl.BlockSpec((BN, D), lambda nb, r: (r * NBLK + nb, 0))
    if with_gidx:
        in_specs += [
            pl.BlockSpec((8, GCOLS), lambda nb, r: (nb, 0)),
            pl.BlockSpec((8, GCOLS), lambda nb, r: (nb, 0)),
        ]
        args += [idx2[0], idx2[1]]
        out_shape = (out_shape,
                     jax.ShapeDtypeStruct((GROWS, GCOLS), jnp.int32))
        out_specs = (out_specs, pl.BlockSpec((8, GCOLS), lambda nb, r: (nb, 0)))

    return pl.pallas_call(
        body,
        grid=(NBLK, R1),
        out_shape=out_shape,
        in_specs=in_specs,
        out_specs=out_specs,
        scratch_shapes=[pltpu.VMEM((R1, D, D), jnp.float32)],
    )(*args)


def _sc_message(Y, gidx, dst3, zeros):
    mesh = plsc.VectorSubcoreMesh(
        core_axis_name="c", subcore_axis_name="s",
        num_cores=NC, num_subcores=NS)

    @functools.partial(
        pl.kernel,
        out_type=jax.ShapeDtypeStruct((NC * N, D), jnp.float32),
        mesh=mesh,
        scratch_types=[
            pltpu.VMEM((NCHUNK, C), jnp.int32),
            pltpu.VMEM((EPW,), jnp.int32),
            pltpu.VMEM((C, D), jnp.float32),
            pltpu.VMEM((C, D), jnp.float32),
            pltpu.VMEM_SHARED((N, D), jnp.float32),
            pltpu.SemaphoreType.DMA,
            pltpu.SemaphoreType.DMA,
        ],
    )
    def k(y_hbm, gidx_hbm, dst_hbm, z_hbm, out_hbm,
          dstm, gidxv, rows0, rows1, agg, sem0, sem1):
        c = lax.axis_index("c")
        s = lax.axis_index("s")
        wid = c * NS + s
        row0 = s * RSTRIDE
        ebase = wid * EPW

        cp_gi = pltpu.async_copy(gidx_hbm.at[pl.ds(ebase, EPW)], gidxv, sem0)
        pltpu.sync_copy(z_hbm.at[pl.ds(row0, RSIZE)],
                        agg.at[pl.ds(row0, RSIZE)])
        pltpu.sync_copy(dst_hbm.at[wid], dstm)
        cp_gi.wait()

        plsc.subcore_barrier()

        cpa = pltpu.async_copy(y_hbm.at[gidxv.at[pl.ds(0, C)]], rows0, sem0)
        cpb = pltpu.async_copy(y_hbm.at[gidxv.at[pl.ds(C, C)]], rows1, sem1)

        def pair(jj, carry):
            j0 = 2 * jj
            cpa.wait()
            pltpu.sync_copy(rows0, agg.at[dstm.at[j0]], add=True)
            pltpu.async_copy(
                y_hbm.at[gidxv.at[pl.ds((j0 + 2) * C, C)]], rows0, sem0)
            cpb.wait()
            pltpu.sync_copy(rows1, agg.at[dstm.at[j0 + 1]], add=True)

            @pl.when(jj < (NCHUNK - 3) // 2)
            def _():
                pltpu.async_copy(
                    y_hbm.at[gidxv.at[pl.ds((j0 + 3) * C, C)]], rows1, sem1)
            return carry

        lax.fori_loop(0, (NCHUNK - 1) // 2, pair, 0)
        cpa.wait()
        pltpu.sync_copy(rows0, agg.at[dstm.at[NCHUNK - 1]], add=True)

        plsc.subcore_barrier()
        pltpu.sync_copy(agg.at[pl.ds(row0, RSIZE)],
                        out_hbm.at[pl.ds(c * N + row0, RSIZE)])

    return k(Y, gidx, dst3, zeros)


def _final_add(P, Y):
    def body(p0_ref, p1_ref, l_ref, o_ref):
        o_ref[...] = p0_ref[...] + p1_ref[...] + l_ref[...]

    return pl.pallas_call(
        body,
        grid=(NBLK,),
        out_shape=jax.ShapeDtypeStruct((N, D), jnp.float32),
        in_specs=[
            pl.BlockSpec((BN, D), lambda i: (i, 0)),
            pl.BlockSpec((BN, D), lambda i: (NBLK + i, 0)),
            pl.BlockSpec((BN, D), lambda i: (R * NBLK + i, 0)),
        ],
        out_specs=pl.BlockSpec((BN, D), lambda i: (i, 0)),
    )(P, P, Y)


@jax.jit
def kernel(emb, edge_index, etype, V, coeff, W_loop):
    src2 = edge_index[0].reshape(GROWS, GCOLS)
    et2 = etype.reshape(GROWS, GCOLS)
    dst3 = edge_index[1].reshape(NW, NCHUNK, C)
    zeros = jnp.zeros((N, D), jnp.float32)
    V_pad = jnp.concatenate([V, W_loop[None]], axis=0)
    coeff_pad = jnp.zeros((R1, NB + 1), jnp.float32)
    coeff_pad = coeff_pad.at[:R, :NB].set(coeff).at[R, NB].set(1.0)

    Y1, gidx2 = _transform([(emb, 0)], coeff_pad, V_pad,
                           idx2=(src2, et2))
    gidx = gidx2.reshape(E)
    P1 = _sc_message(Y1, gidx, dst3, zeros)
    Y2 = _transform(
        [(P1, 0), (P1, NBLK), (Y1, R * NBLK), (emb, 0)], coeff_pad, V_pad)
    P2 = _sc_message(Y2, gidx, dst3, zeros)
    return _final_add(P2, Y2)
